# baseline (speedup 1.0000x reference)
import functools

import jax
import jax.numpy as jnp
from jax import lax
from jax.experimental import pallas as pl
from jax.experimental.pallas import tpu as pltpu
from jax.experimental.pallas import tpu_sc as plsc

_NC = 2
_NS = 16
_NW = _NC * _NS
_LANES = 16
_CHUNK = 256


def _seg_sum_call(n_atoms, d, n_mol):
    m_per_w = n_mol // _NW
    n_col_grp = d // _LANES

    mesh = plsc.VectorSubcoreMesh(
        core_axis_name="c", subcore_axis_name="s",
        num_cores=_NC, num_subcores=_NS)

    @functools.partial(
        pl.kernel,
        out_type=jax.ShapeDtypeStruct((n_mol, d), jnp.float32),
        mesh=mesh,
        scratch_types=[
            pltpu.VMEM((48,), jnp.int32),
            pltpu.VMEM((2 * _CHUNK,), jnp.int32),
            pltpu.VMEM((2, _CHUNK, d), jnp.float32),
            pltpu.VMEM((m_per_w, d), jnp.float32),
            pltpu.SemaphoreType.DMA((2,)),
        ],
    )
    def seg_sum(e_hbm, batch_hbm, bounds_hbm, out_hbm,
                bounds_v, ids_v, rows_v, acc_v, sems):
        wid = lax.axis_index("c") * _NS + lax.axis_index("s")
        pltpu.sync_copy(bounds_hbm, bounds_v)
        bvec = bounds_v[pl.ds(wid, 16)]
        a0 = bvec[0]
        a1 = bvec[1]
        m0 = wid * m_per_w

        zeros = jnp.zeros((_LANES,), jnp.float32)

        def zero_body(i, carry):
            for c in range(n_col_grp):
                acc_v[i, pl.ds(c * _LANES, _LANES)] = zeros
            return carry

        lax.fori_loop(0, m_per_w, zero_body, 0)

        base = a0 & ~7
        n_chunks = (a1 - base + _CHUNK - 1) // _CHUNK

        def chunk_refs(g):
            slot = g % 2
            raw_start = base + g * _CHUNK
            start = pl.multiple_of(
                jnp.minimum(raw_start, n_atoms - _CHUNK), 8)
            return (slot, raw_start, start,
                    batch_hbm.at[pl.ds(start, _CHUNK)],
                    ids_v.at[pl.ds(pl.multiple_of(slot * _CHUNK, 128),
                                   _CHUNK)],
                    e_hbm.at[pl.ds(start, _CHUNK), :],
                    rows_v.at[slot])

        def start_chunk(g):
            slot, _, _, ids_src, ids_dst, row_src, row_dst = chunk_refs(g)
            pltpu.async_copy(ids_src, ids_dst, sems.at[slot])
            pltpu.async_copy(row_src, row_dst, sems.at[slot])

        @pl.when(n_chunks > 0)
        def _():
            start_chunk(0)

        def chunk_body(g, carry):
            @pl.when(g + 1 < n_chunks)
            def _():
                start_chunk(g + 1)

            slot, raw_start, start, ids_src, ids_dst, row_src, row_dst = (
                chunk_refs(g))
            pltpu.make_async_copy(ids_src, ids_dst, sems.at[slot]).wait()
            pltpu.make_async_copy(row_src, row_dst, sems.at[slot]).wait()

            lo = jnp.maximum(a0, raw_start) - start
            hi = jnp.minimum(a1, raw_start + _CHUNK) - start

            @plsc.parallel_loop(lo // _LANES, (hi + _LANES - 1) // _LANES, 1,
                                unroll=4)
            def _(b):
                off = pl.multiple_of(b * _LANES, _LANES)
                ivec = ids_v[pl.ds(
                    pl.multiple_of(slot * _CHUNK + off, _LANES), _LANES)]
                for j in range(_LANES):
                    r = off + j
                    ok = (r >= lo) & (r < hi)
                    seg = jnp.clip(ivec[j] - m0, 0, m_per_w - 1)
                    for c in range(n_col_grp):
                        sl = pl.ds(c * _LANES, _LANES)
                        val = jnp.where(ok, rows_v[slot, r, sl], zeros)
                        plsc.addupdate(acc_v.at[seg, sl], val)

            return carry

        lax.fori_loop(0, n_chunks, chunk_body, 0)
        pltpu.sync_copy(acc_v, out_hbm.at[pl.ds(m0, m_per_w), :])

    return seg_sum


def kernel(E_t, batch):
    n_atoms, d = E_t.shape
    n_mol = 4096
    m_per_w = n_mol // _NW
    mol_starts = jnp.arange(_NW + 1, dtype=jnp.int32) * m_per_w
    bounds = jnp.searchsorted(batch, mol_starts, side="left").astype(jnp.int32)
    bounds = jnp.concatenate([bounds, jnp.zeros((15,), jnp.int32)])
    return _seg_sum_call(n_atoms, d, n_mol)(E_t, batch, bounds)


# speedup vs baseline: 2.2318x; 2.2318x over previous
import functools

import jax
import jax.numpy as jnp
from jax import lax
from jax.experimental import pallas as pl
from jax.experimental.pallas import tpu as pltpu
from jax.experimental.pallas import tpu_sc as plsc

_NC = 2
_NS = 16
_NW = _NC * _NS
_LANES = 16
_CHUNK = 256
_NSUB = _CHUNK // 128
_ACC_ROWS = 2048 + 128


def _seg_sum_call(n_atoms, d, n_mol):
    m_per_core = n_mol // _NC
    m_per_tile = m_per_core // _NS
    zero_rows = _ACC_ROWS // _NS

    mesh = plsc.VectorSubcoreMesh(
        core_axis_name="c", subcore_axis_name="s",
        num_cores=_NC, num_subcores=_NS)

    @functools.partial(
        pl.kernel,
        out_type=jax.ShapeDtypeStruct((n_mol, d), jnp.float32),
        mesh=mesh,
        scratch_types=[
            pltpu.VMEM((48,), jnp.int32),
            pltpu.VMEM((2 * _CHUNK,), jnp.int32),
            pltpu.VMEM((2, _CHUNK, d), jnp.float32),
            pltpu.VMEM((2 * _NSUB, 128), jnp.int32),
            pltpu.VMEM_SHARED((_ACC_ROWS, d), jnp.float32),
            pltpu.SemaphoreType.DMA((2,)),
        ],
    )
    def seg_sum(e_hbm, batch_hbm, bounds_hbm, out_hbm,
                bounds_v, ids_v, rows_v, idx_v, acc_sp, sems):
        core = lax.axis_index("c")
        tid = lax.axis_index("s")
        pltpu.sync_copy(bounds_hbm, bounds_v)
        cvec = bounds_v[pl.ds(core * _NS, 16)]
        cb0 = cvec[0]
        cb1 = bounds_v[pl.ds((core + 1) * _NS, 16)][0]
        clen = cb1 - cb0
        t0 = cb0 + (clen * tid) // _NS
        t1 = cb0 + (clen * (tid + 1)) // _NS
        m0 = core * m_per_core

        zeros = jnp.zeros((_LANES,), jnp.float32)
        lanes = lax.iota(jnp.int32, _LANES)

        @plsc.parallel_loop(0, zero_rows, 1)
        def _(i):
            for c in range(d // _LANES):
                rows_v[0, i, pl.ds(c * _LANES, _LANES)] = zeros

        pltpu.sync_copy(
            rows_v.at[0, pl.ds(0, zero_rows), :],
            acc_sp.at[pl.ds(tid * zero_rows, zero_rows), :])
        plsc.subcore_barrier()

        base = t0 & ~7
        n_chunks = (t1 - base + _CHUNK - 1) // _CHUNK

        def chunk_refs(g):
            slot = g % 2
            raw_start = base + g * _CHUNK
            start = pl.multiple_of(
                jnp.minimum(raw_start, n_atoms - _CHUNK), 8)
            return (slot, raw_start, start,
                    batch_hbm.at[pl.ds(start, _CHUNK)],
                    ids_v.at[pl.ds(pl.multiple_of(slot * _CHUNK, 128),
                                   _CHUNK)],
                    e_hbm.at[pl.ds(start, _CHUNK), :],
                    rows_v.at[slot])

        def start_chunk(g):
            slot, _, _, ids_src, ids_dst, row_src, row_dst = chunk_refs(g)
            pltpu.async_copy(ids_src, ids_dst, sems.at[slot])
            pltpu.async_copy(row_src, row_dst, sems.at[slot])

        @pl.when(n_chunks > 0)
        def _():
            start_chunk(0)

        def chunk_body(g, carry):
            @pl.when(g + 1 < n_chunks)
            def _():
                start_chunk(g + 1)

            slot, raw_start, start, ids_src, ids_dst, row_src, row_dst = (
                chunk_refs(g))
            pltpu.make_async_copy(ids_src, ids_dst, sems.at[slot]).wait()
            pltpu.make_async_copy(row_src, row_dst, sems.at[slot]).wait()

            lo = jnp.maximum(t0, raw_start) - start
            hi = jnp.minimum(t1, raw_start + _CHUNK) - start

            for b in range(_CHUNK // _LANES):
                off = b * _LANES
                ivec = ids_v[pl.ds(
                    pl.multiple_of(slot * _CHUNK + off, _LANES), _LANES)]
                pos = off + lanes
                ok = (pos >= lo) & (pos < hi)
                loc = jnp.where(ok, ivec - m0, m_per_core + lanes)
                sub = b // (128 // _LANES)
                k = b % (128 // _LANES)
                idx_v[slot * _NSUB + sub, pl.ds(k * _LANES, _LANES)] = loc

            for sub in range(_NSUB):
                pltpu.sync_copy(
                    rows_v.at[slot, pl.ds(sub * 128, 128), :],
                    acc_sp.at[idx_v.at[slot * _NSUB + sub]],
                    add=True)
            return carry

        lax.fori_loop(0, n_chunks, chunk_body, 0)
        plsc.subcore_barrier()

        pltpu.sync_copy(
            acc_sp.at[pl.ds(tid * m_per_tile, m_per_tile), :],
            rows_v.at[0, pl.ds(0, m_per_tile), :])
        pltpu.sync_copy(
            rows_v.at[0, pl.ds(0, m_per_tile), :],
            out_hbm.at[pl.ds(m0 + tid * m_per_tile, m_per_tile), :])

    return seg_sum


def kernel(E_t, batch):
    n_atoms, d = E_t.shape
    n_mol = 4096
    m_per_w = n_mol // _NW
    mol_starts = jnp.arange(_NW + 1, dtype=jnp.int32) * m_per_w
    bounds = jnp.searchsorted(batch, mol_starts, side="left").astype(jnp.int32)
    bounds = jnp.concatenate([bounds, jnp.zeros((15,), jnp.int32)])
    return _seg_sum_call(n_atoms, d, n_mol)(E_t, batch, bounds)


# speedup vs baseline: 3.2310x; 1.4477x over previous
import functools

import jax
import jax.numpy as jnp
from jax import lax
from jax.experimental import pallas as pl
from jax.experimental.pallas import tpu as pltpu
from jax.experimental.pallas import tpu_sc as plsc

_NC = 2
_NS = 16
_NW = _NC * _NS
_LANES = 16
_CHUNK = 256
_NSUB = _CHUNK // 128
_ACC_ROWS = 2048 + 128


def _seg_sum_call(n_atoms, d, n_mol):
    m_per_core = n_mol // _NC
    m_per_tile = m_per_core // _NS
    zero_rows = _ACC_ROWS // _NS

    mesh = plsc.VectorSubcoreMesh(
        core_axis_name="c", subcore_axis_name="s",
        num_cores=_NC, num_subcores=_NS)

    @functools.partial(
        pl.kernel,
        out_type=jax.ShapeDtypeStruct((n_mol, d), jnp.float32),
        mesh=mesh,
        scratch_types=[
            pltpu.VMEM((48,), jnp.int32),
            pltpu.VMEM((2 * _CHUNK,), jnp.int32),
            pltpu.VMEM((2, _CHUNK, d), jnp.float32),
            pltpu.VMEM((_ACC_ROWS // _NS, d), jnp.float32),
            pltpu.VMEM((2 * _NSUB, 128), jnp.int32),
            pltpu.VMEM_SHARED((_ACC_ROWS, d), jnp.float32),
            pltpu.SemaphoreType.DMA((2,)),
        ],
    )
    def seg_sum(e_hbm, batch_hbm, bounds_hbm, out_hbm,
                bounds_v, ids_v, rows_v, zero_v, idx_v, acc_sp, sems):
        core = lax.axis_index("c")
        tid = lax.axis_index("s")
        pltpu.sync_copy(bounds_hbm, bounds_v)
        cvec = bounds_v[pl.ds(core * _NS, 16)]
        cb0 = cvec[0]
        cb1 = bounds_v[pl.ds((core + 1) * _NS, 16)][0]
        clen = cb1 - cb0
        t0 = cb0 + (clen * tid) // _NS
        t1 = cb0 + (clen * (tid + 1)) // _NS
        m0 = core * m_per_core

        zeros = jnp.zeros((_LANES,), jnp.float32)
        lanes = lax.iota(jnp.int32, _LANES)

        base = t0 & ~7
        n_chunks = (t1 - base + _CHUNK - 1) // _CHUNK

        def chunk_refs(g):
            slot = g % 2
            raw_start = base + g * _CHUNK
            start = pl.multiple_of(
                jnp.minimum(raw_start, n_atoms - _CHUNK), 8)
            return (slot, raw_start, start,
                    batch_hbm.at[pl.ds(start, _CHUNK)],
                    ids_v.at[pl.ds(pl.multiple_of(slot * _CHUNK, 128),
                                   _CHUNK)],
                    e_hbm.at[pl.ds(start, _CHUNK), :],
                    rows_v.at[slot])

        def start_chunk(g):
            slot, _, _, ids_src, ids_dst, row_src, row_dst = chunk_refs(g)
            pltpu.async_copy(ids_src, ids_dst, sems.at[slot])
            pltpu.async_copy(row_src, row_dst, sems.at[slot])

        @pl.when(n_chunks > 0)
        def _():
            start_chunk(0)

        @plsc.parallel_loop(0, zero_rows, 1)
        def _(i):
            for c in range(d // _LANES):
                zero_v[i, pl.ds(c * _LANES, _LANES)] = zeros

        pltpu.sync_copy(
            zero_v, acc_sp.at[pl.ds(tid * zero_rows, zero_rows), :])
        plsc.subcore_barrier()

        def chunk_body(g, carry):
            @pl.when(g + 1 < n_chunks)
            def _():
                start_chunk(g + 1)

            slot, raw_start, start, ids_src, ids_dst, row_src, row_dst = (
                chunk_refs(g))
            pltpu.make_async_copy(ids_src, ids_dst, sems.at[slot]).wait()
            pltpu.make_async_copy(row_src, row_dst, sems.at[slot]).wait()

            lo = jnp.maximum(t0, raw_start) - start
            hi = jnp.minimum(t1, raw_start + _CHUNK) - start

            for b in range(_CHUNK // _LANES):
                off = b * _LANES
                ivec = ids_v[pl.ds(
                    pl.multiple_of(slot * _CHUNK + off, _LANES), _LANES)]
                pos = off + lanes
                ok = (pos >= lo) & (pos < hi)
                loc = jnp.where(ok, ivec - m0, m_per_core + lanes)
                sub = b // (128 // _LANES)
                k = b % (128 // _LANES)
                idx_v[slot * _NSUB + sub, pl.ds(k * _LANES, _LANES)] = loc

            for sub in range(_NSUB):
                pltpu.sync_copy(
                    rows_v.at[slot, pl.ds(sub * 128, 128), :],
                    acc_sp.at[idx_v.at[slot * _NSUB + sub]],
                    add=True)
            return carry

        lax.fori_loop(0, n_chunks, chunk_body, 0)
        plsc.subcore_barrier()

        pltpu.sync_copy(
            acc_sp.at[pl.ds(tid * m_per_tile, m_per_tile), :],
            rows_v.at[0, pl.ds(0, m_per_tile), :])
        pltpu.sync_copy(
            rows_v.at[0, pl.ds(0, m_per_tile), :],
            out_hbm.at[pl.ds(m0 + tid * m_per_tile, m_per_tile), :])

    return seg_sum


def kernel(E_t, batch):
    n_atoms, d = E_t.shape
    n_mol = 4096
    m_per_w = n_mol // _NW
    mol_starts = jnp.arange(_NW + 1, dtype=jnp.int32) * m_per_w
    bounds = jnp.searchsorted(batch, mol_starts, side="left",
                              method="compare_all").astype(jnp.int32)
    bounds = jnp.concatenate([bounds, jnp.zeros((15,), jnp.int32)])
    return _seg_sum_call(n_atoms, d, n_mol)(E_t, batch, bounds)
